# baseline (speedup 1.0000x reference)
import functools

import jax
import jax.numpy as jnp
from jax import lax
from jax.experimental import pallas as pl
from jax.experimental.pallas import tpu as pltpu
from jax.experimental.pallas import tpu_sc as plsc

VOCAB = 100000
DIM = 300
BATCH = 1024
MAX_NORM = 0.15


_NC, _NS = 2, 16
_NW = _NC * _NS
_B_PER_W = BATCH // _NW
_LANES = 16


def _sc_gather(table, idx):
    mesh = plsc.VectorSubcoreMesh(core_axis_name="c", subcore_axis_name="s")

    @functools.partial(
        pl.kernel,
        mesh=mesh,
        out_type=jax.ShapeDtypeStruct((BATCH, DIM), jnp.float32),
        scratch_types=[
            pltpu.VMEM((_B_PER_W,), jnp.int32),
            pltpu.VMEM((_B_PER_W, DIM), jnp.float32),
            pltpu.SemaphoreType.DMA,
        ],
    )
    def gather_kernel(table_hbm, idx_hbm, out_hbm, idx_v, rows_v, sem):
        wid = lax.axis_index("s") * _NC + lax.axis_index("c")
        base = wid * _B_PER_W
        pltpu.sync_copy(idx_hbm.at[pl.ds(base, _B_PER_W)], idx_v)
        copies = []
        for c in range(_B_PER_W // _LANES):
            chunk = idx_v[pl.ds(c * _LANES, _LANES)]
            for k in range(_LANES):
                j = c * _LANES + k
                row = chunk[k]
                cp = pltpu.make_async_copy(
                    table_hbm.at[pl.ds(row, 1)], rows_v.at[pl.ds(j, 1)], sem
                )
                cp.start()
                copies.append(cp)
        for cp in copies:
            cp.wait()
        pltpu.sync_copy(rows_v, out_hbm.at[pl.ds(base, _B_PER_W)])

    return gather_kernel(table, idx)


SW = 8192
NS_FULL = 12
TAIL = VOCAB - NS_FULL * SW
TAIL_W = 2048
RB = 256
NR = BATCH // RB
N_STRIPES = NS_FULL + 1
B_PAD = N_STRIPES * SW


def _w_read(w_ref, wb, rsem, s):
    return pltpu.make_async_copy(
        w_ref.at[pl.ds(s * SW, SW)], wb.at[s % 2], rsem.at[s % 2]
    )


def _w_read_tail(w_ref, wb, rsem):
    return pltpu.make_async_copy(
        w_ref.at[pl.ds(NS_FULL * SW, TAIL)],
        wb.at[NS_FULL % 2].at[pl.ds(0, TAIL)],
        rsem.at[NS_FULL % 2],
    )


def _o_write(out_ref, ob, wsem, s, r, slot):
    return pltpu.make_async_copy(
        ob.at[slot],
        out_ref.at[pl.ds(r * RB, RB), pl.ds(s * SW, SW)],
        wsem.at[slot],
    )


def _o_write_tail(tail_ref, ob, wsem, r, slot):
    return pltpu.make_async_copy(
        ob.at[slot].at[:, pl.ds(0, TAIL_W)],
        tail_ref.at[pl.ds(r * RB, RB)],
        wsem.at[slot],
    )


def _mm_body(x_ref, w_ref, b_ref, out_ref, tail_ref, xs, bv, wb, ob, csem, rsem, wsem):
    s = pl.program_id(0)
    r = pl.program_id(1)
    t = s * NR + r
    slot = t % 2

    @pl.when(t == 0)
    def _prologue():
        _w_read(w_ref, wb, rsem, 0).start()
        _w_read(w_ref, wb, rsem, 1).start()
        pltpu.make_async_copy(b_ref, bv, csem).start()
        pltpu.make_async_copy(b_ref, bv, csem).wait()
        x = x_ref[...]
        nrm = jnp.sqrt(jnp.sum(x * x, axis=1, keepdims=True))
        scale = jnp.where(nrm > MAX_NORM, MAX_NORM / (nrm + 1e-7), 1.0)
        xs[...] = x * scale

    @pl.when((r == 0) & (s < NS_FULL))
    def _wait_w():
        _w_read(w_ref, wb, rsem, s).wait()

    @pl.when((r == 0) & (s == NS_FULL))
    def _wait_w_tail():
        _w_read_tail(w_ref, wb, rsem).wait()

    @pl.when((r == 1) & (s >= 1) & (s + 1 < NS_FULL))
    def _prefetch_w():
        _w_read(w_ref, wb, rsem, s + 1).start()

    @pl.when((r == 1) & (s + 1 == NS_FULL))
    def _prefetch_w_tail():
        _w_read_tail(w_ref, wb, rsem).start()

    @pl.when((t >= 2) & (t - 2 < NS_FULL * NR))
    def _free_slot_full():
        pltpu.make_async_copy(
            ob.at[slot], out_ref.at[pl.ds(0, RB), pl.ds(0, SW)], wsem.at[slot]
        ).wait()

    @pl.when(t - 2 >= NS_FULL * NR)
    def _free_slot_tail():
        pltpu.make_async_copy(
            ob.at[slot].at[:, pl.ds(0, TAIL_W)],
            tail_ref.at[pl.ds(0, RB)],
            wsem.at[slot],
        ).wait()

    xr = xs[pl.ds(r * RB, RB), :]

    @pl.when(s < NS_FULL)
    def _compute_full():
        acc = lax.dot_general(
            xr,
            wb[s % 2],
            (((1,), (1,)), ((), ())),
            preferred_element_type=jnp.float32,
        )
        ob[slot] = acc + bv[:, pl.ds(s * SW, SW)]
        _o_write(out_ref, ob, wsem, s, r, slot).start()

    @pl.when(s == NS_FULL)
    def _compute_tail():
        acc = lax.dot_general(
            xr,
            wb.at[s % 2][pl.ds(0, TAIL_W), :],
            (((1,), (1,)), ((), ())),
            preferred_element_type=jnp.float32,
        )
        ob_t = ob.at[slot].at[:, pl.ds(0, TAIL_W)]
        ob_t[...] = acc + bv[:, pl.ds(NS_FULL * SW, TAIL_W)]
        _o_write_tail(tail_ref, ob, wsem, r, slot).start()

    @pl.when(t == N_STRIPES * NR - 1)
    def _drain():
        for d in (1, 0):
            pltpu.make_async_copy(
                ob.at[(t - d) % 2].at[:, pl.ds(0, TAIL_W)],
                tail_ref.at[pl.ds(0, RB)],
                wsem.at[(t - d) % 2],
            ).wait()


def _matmul_main(x, W, b2):
    return pl.pallas_call(
        _mm_body,
        grid=(N_STRIPES, NR),
        in_specs=[
            pl.BlockSpec((BATCH, DIM), lambda s, r: (0, 0)),
            pl.BlockSpec(memory_space=pltpu.MemorySpace.HBM),
            pl.BlockSpec(memory_space=pltpu.MemorySpace.HBM),
        ],
        out_specs=[
            pl.BlockSpec(memory_space=pltpu.MemorySpace.HBM),
            pl.BlockSpec(memory_space=pltpu.MemorySpace.HBM),
        ],
        out_shape=[
            jax.ShapeDtypeStruct((BATCH, VOCAB), jnp.float32),
            jax.ShapeDtypeStruct((BATCH, TAIL_W), jnp.float32),
        ],
        scratch_shapes=[
            pltpu.VMEM((BATCH, DIM), jnp.float32),
            pltpu.VMEM((1, B_PAD), jnp.float32),
            pltpu.VMEM((2, SW, DIM), jnp.float32),
            pltpu.VMEM((2, RB, SW), jnp.float32),
            pltpu.SemaphoreType.DMA,
            pltpu.SemaphoreType.DMA((2,)),
            pltpu.SemaphoreType.DMA((2,)),
        ],
    )(x, W, b2)


def _tail_merge_body(big_ref, tmp_ref, out_ref):
    out_ref[...] = tmp_ref[...]


def _tail_merge(big, tail_tmp):
    n_blk = VOCAB // TAIL_W
    return pl.pallas_call(
        _tail_merge_body,
        grid=(1,),
        in_specs=[
            pl.BlockSpec(memory_space=pltpu.MemorySpace.HBM),
            pl.BlockSpec((BATCH, TAIL_W), lambda i: (0, 0)),
        ],
        out_specs=pl.BlockSpec((BATCH, TAIL_W), lambda i: (0, n_blk)),
        out_shape=jax.ShapeDtypeStruct((BATCH, VOCAB), jnp.float32),
        input_output_aliases={0: 0},
    )(big, tail_tmp)


@jax.jit
def kernel(_inputs, target_table, W, b):
    idx = _inputs.astype(jnp.int32)
    x_raw = _sc_gather(target_table, idx)
    b_pad = jnp.zeros((1, B_PAD), jnp.float32).at[:, :VOCAB].set(b.reshape(1, VOCAB))
    big, tail_tmp = _matmul_main(x_raw, W, b_pad)
    return _tail_merge(big, tail_tmp)


# speedup vs baseline: 2.1955x; 2.1955x over previous
import functools

import jax
import jax.numpy as jnp
from jax import lax
from jax.experimental import pallas as pl
from jax.experimental.pallas import tpu as pltpu
from jax.experimental.pallas import tpu_sc as plsc

VOCAB = 100000
DIM = 300
BATCH = 1024
MAX_NORM = 0.15


_NC, _NS = 2, 16
_NW = _NC * _NS
_B_PER_W = BATCH // _NW
_LANES = 16


def _sc_gather(table, idx):
    mesh = plsc.VectorSubcoreMesh(core_axis_name="c", subcore_axis_name="s")

    @functools.partial(
        pl.kernel,
        mesh=mesh,
        out_type=jax.ShapeDtypeStruct((BATCH, DIM), jnp.float32),
        scratch_types=[
            pltpu.VMEM((_B_PER_W,), jnp.int32),
            pltpu.VMEM((_B_PER_W, DIM), jnp.float32),
            pltpu.SemaphoreType.DMA,
        ],
    )
    def gather_kernel(table_hbm, idx_hbm, out_hbm, idx_v, rows_v, sem):
        wid = lax.axis_index("s") * _NC + lax.axis_index("c")
        base = wid * _B_PER_W
        pltpu.sync_copy(idx_hbm.at[pl.ds(base, _B_PER_W)], idx_v)
        copies = []
        for c in range(_B_PER_W // _LANES):
            chunk = idx_v[pl.ds(c * _LANES, _LANES)]
            for k in range(_LANES):
                j = c * _LANES + k
                row = chunk[k]
                cp = pltpu.make_async_copy(
                    table_hbm.at[pl.ds(row, 1)], rows_v.at[pl.ds(j, 1)], sem
                )
                cp.start()
                copies.append(cp)
        for cp in copies:
            cp.wait()
        pltpu.sync_copy(rows_v, out_hbm.at[pl.ds(base, _B_PER_W)])

    return gather_kernel(table, idx)


SW = 3584
NS_FULL = 27
TAIL = 99968 - NS_FULL * SW
SLIVER = VOCAB - 99968
TAIL_ALL = TAIL + SLIVER
N_STEPS = NS_FULL + 1


def _w_read(wt_ref, bc_ref, wb, bb, rsem, s):
    return (
        pltpu.make_async_copy(
            wt_ref.at[:, pl.ds(s * SW, SW)], wb.at[s % 2], rsem.at[s % 2]
        ),
        pltpu.make_async_copy(
            bc_ref.at[pl.ds(s * SW, SW)], bb.at[s % 2], rsem.at[s % 2]
        ),
    )


def _w_read_tail(wt_ref, bc_ref, wb, bb, rsem):
    s = NS_FULL
    return (
        pltpu.make_async_copy(
            wt_ref.at[:, pl.ds(s * SW, TAIL)],
            wb.at[s % 2].at[:, pl.ds(0, TAIL)],
            rsem.at[s % 2],
        ),
        pltpu.make_async_copy(
            bc_ref.at[pl.ds(s * SW, TAIL)],
            bb.at[s % 2].at[pl.ds(0, TAIL)],
            rsem.at[s % 2],
        ),
    )


def _o_write(out_ref, ob, wsem, s, slot):
    return pltpu.make_async_copy(
        ob.at[slot], out_ref.at[pl.ds(s * SW, SW)], wsem.at[slot]
    )


def _o_write_tail(out_ref, ob, wsem, slot):
    return pltpu.make_async_copy(
        ob.at[slot].at[pl.ds(0, TAIL_ALL)],
        out_ref.at[pl.ds(NS_FULL * SW, TAIL_ALL)],
        wsem.at[slot],
    )


def _mm_body(x_ref, wt_ref, bc_ref, w32_ref, b32_ref, out_ref, xs, wb, bb, ob, rsem, wsem):
    t = pl.program_id(0)
    slot = t % 2

    @pl.when(t == 0)
    def _prologue():
        for cp in _w_read(wt_ref, bc_ref, wb, bb, rsem, 0):
            cp.start()
        for cp in _w_read(wt_ref, bc_ref, wb, bb, rsem, 1):
            cp.start()
        x = x_ref[...]
        nrm = jnp.sqrt(jnp.sum(x * x, axis=1, keepdims=True))
        scale = jnp.where(nrm > MAX_NORM, MAX_NORM / (nrm + 1e-7), 1.0)
        xs[...] = jnp.transpose(x * scale)

    @pl.when(t < NS_FULL)
    def _wait_w():
        for cp in _w_read(wt_ref, bc_ref, wb, bb, rsem, t):
            cp.wait()

    @pl.when(t == NS_FULL)
    def _wait_w_tail():
        for cp in _w_read_tail(wt_ref, bc_ref, wb, bb, rsem):
            cp.wait()

    @pl.when((t >= 1) & (t + 1 < NS_FULL))
    def _prefetch_w():
        for cp in _w_read(wt_ref, bc_ref, wb, bb, rsem, t + 1):
            cp.start()

    @pl.when(t + 1 == NS_FULL)
    def _prefetch_w_tail():
        for cp in _w_read_tail(wt_ref, bc_ref, wb, bb, rsem):
            cp.start()

    @pl.when(t >= 2)
    def _free_slot():
        pltpu.make_async_copy(
            ob.at[slot], out_ref.at[pl.ds(0, SW)], wsem.at[slot]
        ).wait()

    xsv = xs[...]

    @pl.when(t < NS_FULL)
    def _compute_full():
        acc = lax.dot_general(
            wb[slot],
            xsv,
            (((0,), (0,)), ((), ())),
            preferred_element_type=jnp.float32,
        )
        ob[slot] = acc + bb[slot]
        _o_write(out_ref, ob, wsem, t, slot).start()

    @pl.when(t == NS_FULL)
    def _compute_tail():
        acc = lax.dot_general(
            wb.at[slot][:, pl.ds(0, TAIL)],
            xsv,
            (((0,), (0,)), ((), ())),
            preferred_element_type=jnp.float32,
        )
        acc32 = lax.dot_general(
            w32_ref[...],
            xsv,
            (((0,), (0,)), ((), ())),
            preferred_element_type=jnp.float32,
        )
        ob_t = ob.at[slot].at[pl.ds(0, TAIL)]
        ob_t[...] = acc + bb.at[slot][pl.ds(0, TAIL)]
        ob_s = ob.at[slot].at[pl.ds(TAIL, SLIVER)]
        ob_s[...] = acc32 + b32_ref[...]
        _o_write_tail(out_ref, ob, wsem, slot).start()
        pltpu.make_async_copy(
            ob.at[(t - 1) % 2], out_ref.at[pl.ds(0, SW)], wsem.at[(t - 1) % 2]
        ).wait()
        _o_write_tail(out_ref, ob, wsem, slot).wait()


def _matmul_t(x, WT, b_col, W32, b32):
    return pl.pallas_call(
        _mm_body,
        grid=(N_STEPS,),
        in_specs=[
            pl.BlockSpec((BATCH, DIM), lambda t: (0, 0)),
            pl.BlockSpec(memory_space=pltpu.MemorySpace.HBM),
            pl.BlockSpec(memory_space=pltpu.MemorySpace.HBM),
            pl.BlockSpec((DIM, SLIVER), lambda t: (0, 0)),
            pl.BlockSpec((SLIVER, 1), lambda t: (0, 0)),
        ],
        out_specs=pl.BlockSpec(memory_space=pltpu.MemorySpace.HBM),
        out_shape=jax.ShapeDtypeStruct((VOCAB, BATCH), jnp.float32),
        scratch_shapes=[
            pltpu.VMEM((DIM, BATCH), jnp.float32),
            pltpu.VMEM((2, DIM, SW), jnp.float32),
            pltpu.VMEM((2, SW, 1), jnp.float32),
            pltpu.VMEM((2, SW, BATCH), jnp.float32),
            pltpu.SemaphoreType.DMA((2,)),
            pltpu.SemaphoreType.DMA((2,)),
        ],
    )(x, WT, b_col, W32, b32)


@jax.jit
def kernel(_inputs, target_table, W, b):
    idx = _inputs.astype(jnp.int32)
    x_raw = _sc_gather(target_table, idx)
    WT = W.T
    b_col = b.reshape(VOCAB, 1)
    W32 = W[99968:].T
    b32 = b[99968:].reshape(SLIVER, 1)
    outT = _matmul_t(x_raw, WT, b_col, W32, b32)
    return outT.T


# speedup vs baseline: 2.5795x; 1.1749x over previous
import functools

import jax
import jax.numpy as jnp
from jax import lax
from jax.experimental import pallas as pl
from jax.experimental.pallas import tpu as pltpu
from jax.experimental.pallas import tpu_sc as plsc

VOCAB = 100000
DIM = 300
BATCH = 1024
MAX_NORM = 0.15


_NC, _NS = 2, 16
_NW = _NC * _NS
_B_PER_W = BATCH // _NW
_LANES = 16


def _sc_gather(table, idx):
    mesh = plsc.VectorSubcoreMesh(core_axis_name="c", subcore_axis_name="s")

    @functools.partial(
        pl.kernel,
        mesh=mesh,
        out_type=jax.ShapeDtypeStruct((BATCH, DIM), jnp.float32),
        scratch_types=[
            pltpu.VMEM((_B_PER_W,), jnp.int32),
            pltpu.VMEM((_B_PER_W, DIM), jnp.float32),
            pltpu.SemaphoreType.DMA,
        ],
    )
    def gather_kernel(table_hbm, idx_hbm, out_hbm, idx_v, rows_v, sem):
        wid = lax.axis_index("s") * _NC + lax.axis_index("c")
        base = wid * _B_PER_W
        pltpu.sync_copy(idx_hbm.at[pl.ds(base, _B_PER_W)], idx_v)
        copies = []
        for c in range(_B_PER_W // _LANES):
            chunk = idx_v[pl.ds(c * _LANES, _LANES)]
            for k in range(_LANES):
                j = c * _LANES + k
                row = chunk[k]
                cp = pltpu.make_async_copy(
                    table_hbm.at[pl.ds(row, 1)], rows_v.at[pl.ds(j, 1)], sem
                )
                cp.start()
                copies.append(cp)
        for cp in copies:
            cp.wait()
        pltpu.sync_copy(rows_v, out_hbm.at[pl.ds(base, _B_PER_W)])

    return gather_kernel(table, idx)


SW = 3584
NS_FULL = 27
TAIL = 99968 - NS_FULL * SW
SLIVER = VOCAB - 99968
TAIL_ALL = TAIL + SLIVER
N_STEPS = NS_FULL + 1


def _w_read(wt_ref, bc_ref, wb, bb, rsem, s):
    return (
        pltpu.make_async_copy(
            wt_ref.at[:, pl.ds(s * SW, SW)], wb.at[s % 2], rsem.at[s % 2]
        ),
        pltpu.make_async_copy(
            bc_ref.at[:, pl.ds(s * SW, SW)], bb.at[s % 2], rsem.at[s % 2]
        ),
    )


def _w_read_tail(wt_ref, bc_ref, wb, bb, rsem):
    s = NS_FULL
    return (
        pltpu.make_async_copy(
            wt_ref.at[:, pl.ds(s * SW, TAIL)],
            wb.at[s % 2].at[:, pl.ds(0, TAIL)],
            rsem.at[s % 2],
        ),
        pltpu.make_async_copy(
            bc_ref.at[:, pl.ds(s * SW, TAIL)],
            bb.at[s % 2].at[:, pl.ds(0, TAIL)],
            rsem.at[s % 2],
        ),
    )


def _o_write(out_ref, ob, wsem, s, slot):
    return pltpu.make_async_copy(
        ob.at[slot], out_ref.at[pl.ds(s * SW, SW)], wsem.at[slot]
    )


def _o_write_tail(out_ref, ob, wsem, slot):
    return pltpu.make_async_copy(
        ob.at[slot].at[pl.ds(0, TAIL_ALL)],
        out_ref.at[pl.ds(NS_FULL * SW, TAIL_ALL)],
        wsem.at[slot],
    )


def _mm_body(x_ref, wt_ref, bc_ref, w32_ref, b32_ref, out_ref, xs, wb, bb, ob, rsem, wsem):
    t = pl.program_id(0)
    slot = t % 2

    @pl.when(t == 0)
    def _prologue():
        for cp in _w_read(wt_ref, bc_ref, wb, bb, rsem, 0):
            cp.start()
        for cp in _w_read(wt_ref, bc_ref, wb, bb, rsem, 1):
            cp.start()
        x = x_ref[...]
        nrm = jnp.sqrt(jnp.sum(x * x, axis=1, keepdims=True))
        scale = jnp.where(nrm > MAX_NORM, MAX_NORM / (nrm + 1e-7), 1.0)
        xs[...] = jnp.transpose(x * scale)

    @pl.when(t < NS_FULL)
    def _wait_w():
        for cp in _w_read(wt_ref, bc_ref, wb, bb, rsem, t):
            cp.wait()

    @pl.when(t == NS_FULL)
    def _wait_w_tail():
        for cp in _w_read_tail(wt_ref, bc_ref, wb, bb, rsem):
            cp.wait()

    @pl.when((t >= 1) & (t + 1 < NS_FULL))
    def _prefetch_w():
        for cp in _w_read(wt_ref, bc_ref, wb, bb, rsem, t + 1):
            cp.start()

    @pl.when(t + 1 == NS_FULL)
    def _prefetch_w_tail():
        for cp in _w_read_tail(wt_ref, bc_ref, wb, bb, rsem):
            cp.start()

    @pl.when(t >= 2)
    def _free_slot():
        pltpu.make_async_copy(
            ob.at[slot], out_ref.at[pl.ds(0, SW)], wsem.at[slot]
        ).wait()

    xsv = xs[...]

    @pl.when(t < NS_FULL)
    def _compute_full():
        acc = lax.dot_general(
            wb[slot],
            xsv,
            (((0,), (0,)), ((), ())),
            preferred_element_type=jnp.float32,
        )
        ob[slot] = acc + jnp.transpose(bb[slot])
        _o_write(out_ref, ob, wsem, t, slot).start()

    @pl.when(t == NS_FULL)
    def _compute_tail():
        acc = lax.dot_general(
            wb.at[slot][:, pl.ds(0, TAIL)],
            xsv,
            (((0,), (0,)), ((), ())),
            preferred_element_type=jnp.float32,
        )
        acc32 = lax.dot_general(
            w32_ref[...],
            xsv,
            (((0,), (0,)), ((), ())),
            preferred_element_type=jnp.float32,
        )
        ob_t = ob.at[slot].at[pl.ds(0, TAIL)]
        ob_t[...] = acc + jnp.transpose(bb.at[slot][:, pl.ds(0, TAIL)])
        ob_s = ob.at[slot].at[pl.ds(TAIL, SLIVER)]
        ob_s[...] = acc32 + jnp.transpose(b32_ref[...])
        _o_write_tail(out_ref, ob, wsem, slot).start()
        pltpu.make_async_copy(
            ob.at[(t - 1) % 2], out_ref.at[pl.ds(0, SW)], wsem.at[(t - 1) % 2]
        ).wait()
        _o_write_tail(out_ref, ob, wsem, slot).wait()


def _matmul_t(x, WT, b_col, W32, b32):
    return pl.pallas_call(
        _mm_body,
        grid=(N_STEPS,),
        in_specs=[
            pl.BlockSpec((BATCH, DIM), lambda t: (0, 0)),
            pl.BlockSpec(memory_space=pltpu.MemorySpace.HBM),
            pl.BlockSpec(memory_space=pltpu.MemorySpace.HBM),
            pl.BlockSpec((DIM, SLIVER), lambda t: (0, 0)),
            pl.BlockSpec((1, SLIVER), lambda t: (0, 0)),
        ],
        out_specs=pl.BlockSpec(memory_space=pltpu.MemorySpace.HBM),
        out_shape=jax.ShapeDtypeStruct((VOCAB, BATCH), jnp.float32),
        scratch_shapes=[
            pltpu.VMEM((DIM, BATCH), jnp.float32),
            pltpu.VMEM((2, DIM, SW), jnp.float32),
            pltpu.VMEM((2, 1, SW), jnp.float32),
            pltpu.VMEM((2, SW, BATCH), jnp.float32),
            pltpu.SemaphoreType.DMA((2,)),
            pltpu.SemaphoreType.DMA((2,)),
        ],
    )(x, WT, b_col, W32, b32)


@jax.jit
def kernel(_inputs, target_table, W, b):
    idx = _inputs.astype(jnp.int32)
    x_raw = _sc_gather(target_table, idx)
    WT = W.T
    b_col = b.reshape(1, VOCAB)
    W32 = W[99968:].T
    b32 = b[99968:].reshape(1, SLIVER)
    outT = _matmul_t(x_raw, WT, b_col, W32, b32)
    return outT.T


# speedup vs baseline: 2.5859x; 1.0025x over previous
import functools

import jax
import jax.numpy as jnp
from jax import lax
from jax.experimental import pallas as pl
from jax.experimental.pallas import tpu as pltpu
from jax.experimental.pallas import tpu_sc as plsc

VOCAB = 100000
DIM = 300
BATCH = 1024
MAX_NORM = 0.15


_NC, _NS = 2, 16
_NW = _NC * _NS
_B_PER_W = BATCH // _NW
_LANES = 16


def _sc_gather(table, idx):
    mesh = plsc.VectorSubcoreMesh(core_axis_name="c", subcore_axis_name="s")

    @functools.partial(
        pl.kernel,
        mesh=mesh,
        out_type=jax.ShapeDtypeStruct((BATCH, DIM), jnp.float32),
        scratch_types=[
            pltpu.VMEM((_B_PER_W,), jnp.int32),
            pltpu.VMEM((_B_PER_W, DIM), jnp.float32),
            pltpu.SemaphoreType.DMA,
        ],
    )
    def gather_kernel(table_hbm, idx_hbm, out_hbm, idx_v, rows_v, sem):
        wid = lax.axis_index("s") * _NC + lax.axis_index("c")
        base = wid * _B_PER_W
        pltpu.sync_copy(idx_hbm.at[pl.ds(base, _B_PER_W)], idx_v)
        copies = []
        for c in range(_B_PER_W // _LANES):
            chunk = idx_v[pl.ds(c * _LANES, _LANES)]
            for k in range(_LANES):
                j = c * _LANES + k
                row = chunk[k]
                cp = pltpu.make_async_copy(
                    table_hbm.at[pl.ds(row, 1)], rows_v.at[pl.ds(j, 1)], sem
                )
                cp.start()
                copies.append(cp)
        for cp in copies:
            cp.wait()
        pltpu.sync_copy(rows_v, out_hbm.at[pl.ds(base, _B_PER_W)])

    return gather_kernel(table, idx)


SW = 3584
NS_FULL = 27
TAIL = 99968 - NS_FULL * SW
SLIVER = VOCAB - 99968
TAIL_ALL = TAIL + SLIVER
N_STEPS = NS_FULL + 1


def _w_read(wt_ref, bc_ref, wb, bb, rsem, s):
    return (
        pltpu.make_async_copy(
            wt_ref.at[:, pl.ds(s * SW, SW)], wb.at[s % 2], rsem.at[s % 2]
        ),
        pltpu.make_async_copy(
            bc_ref.at[:, pl.ds(s * SW, SW)], bb.at[s % 2], rsem.at[s % 2]
        ),
    )


def _w_read_tail(wt_ref, bc_ref, wb, bb, rsem):
    s = NS_FULL
    return (
        pltpu.make_async_copy(
            wt_ref.at[:, pl.ds(s * SW, TAIL)],
            wb.at[s % 2].at[:, pl.ds(0, TAIL)],
            rsem.at[s % 2],
        ),
        pltpu.make_async_copy(
            bc_ref.at[:, pl.ds(s * SW, TAIL)],
            bb.at[s % 2].at[:, pl.ds(0, TAIL)],
            rsem.at[s % 2],
        ),
    )


def _o_write(out_ref, ob, wsem, s, slot):
    return pltpu.make_async_copy(
        ob.at[slot], out_ref.at[pl.ds(s * SW, SW)], wsem.at[slot]
    )


def _o_write_tail(out_ref, ob, wsem, slot):
    return pltpu.make_async_copy(
        ob.at[slot].at[pl.ds(0, TAIL_ALL)],
        out_ref.at[pl.ds(NS_FULL * SW, TAIL_ALL)],
        wsem.at[slot],
    )


def _mm_body(x_ref, wt_ref, bc_ref, w32_ref, b32_ref, out_ref, xs, wb, bb, ob, rsem, wsem):
    t = pl.program_id(0)
    slot = t % 2

    @pl.when(t == 0)
    def _prologue():
        for cp in _w_read(wt_ref, bc_ref, wb, bb, rsem, 0):
            cp.start()
        for cp in _w_read(wt_ref, bc_ref, wb, bb, rsem, 1):
            cp.start()
        x = x_ref[...]
        nrm = jnp.sqrt(jnp.sum(x * x, axis=1, keepdims=True))
        scale = jnp.where(nrm > MAX_NORM, MAX_NORM / (nrm + 1e-7), 1.0)
        xs[...] = jnp.transpose(x * scale)

    @pl.when(t < NS_FULL)
    def _wait_w():
        for cp in _w_read(wt_ref, bc_ref, wb, bb, rsem, t):
            cp.wait()

    @pl.when(t == NS_FULL)
    def _wait_w_tail():
        for cp in _w_read_tail(wt_ref, bc_ref, wb, bb, rsem):
            cp.wait()

    @pl.when((t >= 1) & (t + 1 < NS_FULL))
    def _prefetch_w():
        for cp in _w_read(wt_ref, bc_ref, wb, bb, rsem, t + 1):
            cp.start()

    @pl.when(t + 1 == NS_FULL)
    def _prefetch_w_tail():
        for cp in _w_read_tail(wt_ref, bc_ref, wb, bb, rsem):
            cp.start()

    @pl.when(t >= 2)
    def _free_slot():
        pltpu.make_async_copy(
            ob.at[slot], out_ref.at[pl.ds(0, SW)], wsem.at[slot]
        ).wait()

    xsv = xs[...]

    @pl.when(t < NS_FULL)
    def _compute_full():
        acc = lax.dot_general(
            wb[slot].astype(jnp.bfloat16),
            xsv.astype(jnp.bfloat16),
            (((0,), (0,)), ((), ())),
            preferred_element_type=jnp.float32,
        )
        ob[slot] = acc + jnp.transpose(bb[slot])
        _o_write(out_ref, ob, wsem, t, slot).start()

    @pl.when(t == NS_FULL)
    def _compute_tail():
        acc = lax.dot_general(
            wb.at[slot][:, pl.ds(0, TAIL)],
            xsv,
            (((0,), (0,)), ((), ())),
            preferred_element_type=jnp.float32,
        )
        acc32 = lax.dot_general(
            w32_ref[...],
            xsv,
            (((0,), (0,)), ((), ())),
            preferred_element_type=jnp.float32,
        )
        ob_t = ob.at[slot].at[pl.ds(0, TAIL)]
        ob_t[...] = acc + jnp.transpose(bb.at[slot][:, pl.ds(0, TAIL)])
        ob_s = ob.at[slot].at[pl.ds(TAIL, SLIVER)]
        ob_s[...] = acc32 + jnp.transpose(b32_ref[...])
        _o_write_tail(out_ref, ob, wsem, slot).start()
        pltpu.make_async_copy(
            ob.at[(t - 1) % 2], out_ref.at[pl.ds(0, SW)], wsem.at[(t - 1) % 2]
        ).wait()
        _o_write_tail(out_ref, ob, wsem, slot).wait()


def _matmul_t(x, WT, b_col, W32, b32):
    return pl.pallas_call(
        _mm_body,
        grid=(N_STEPS,),
        in_specs=[
            pl.BlockSpec((BATCH, DIM), lambda t: (0, 0)),
            pl.BlockSpec(memory_space=pltpu.MemorySpace.HBM),
            pl.BlockSpec(memory_space=pltpu.MemorySpace.HBM),
            pl.BlockSpec((DIM, SLIVER), lambda t: (0, 0)),
            pl.BlockSpec((1, SLIVER), lambda t: (0, 0)),
        ],
        out_specs=pl.BlockSpec(memory_space=pltpu.MemorySpace.HBM),
        out_shape=jax.ShapeDtypeStruct((VOCAB, BATCH), jnp.float32),
        scratch_shapes=[
            pltpu.VMEM((DIM, BATCH), jnp.float32),
            pltpu.VMEM((2, DIM, SW), jnp.float32),
            pltpu.VMEM((2, 1, SW), jnp.float32),
            pltpu.VMEM((2, SW, BATCH), jnp.float32),
            pltpu.SemaphoreType.DMA((2,)),
            pltpu.SemaphoreType.DMA((2,)),
        ],
    )(x, WT, b_col, W32, b32)


@jax.jit
def kernel(_inputs, target_table, W, b):
    idx = _inputs.astype(jnp.int32)
    x_raw = _sc_gather(target_table, idx)
    WT = W.T
    b_col = b.reshape(1, VOCAB)
    W32 = W[99968:].T
    b32 = b[99968:].reshape(1, SLIVER)
    outT = _matmul_t(x_raw, WT, b_col, W32, b32)
    return outT.T
